# baseline (speedup 1.0000x reference)
import jax
import jax.numpy as jnp
from jax.experimental import pallas as pl
from jax.experimental.pallas import tpu as pltpu

_K = 8192
_D = 32
_N = 4096
_R = 128
_G = _N // _R


def _wn_body(w_ref, wn_ref):
    w = w_ref[...]
    wn_ref[...] = jnp.sum(w * w, axis=1)[None, :]


def _vq_body(x_ref, w_ref, wn_ref, s_ref, enc_ref, q_ref, pprobs_ref,
             ploss_ref):
    x = x_ref[...]
    w = w_ref[...]
    rn = jnp.sum(x * x, axis=1, keepdims=True)
    wn = wn_ref[...]
    mm = jax.lax.dot_general(x, w, (((1,), (1,)), ((), ())),
                             preferred_element_type=jnp.float32)
    d = (rn + wn) - 2.0 * mm

    md = jnp.min(d, axis=1, keepdims=True)
    iota = jax.lax.broadcasted_iota(jnp.int32, d.shape, 1)
    idx = jnp.min(jnp.where(d == md, iota, jnp.int32(_K)),
                  axis=1, keepdims=True)

    mds = md + s_ref[0, 0]
    inv = 1.0 / mds
    norm = jnp.sqrt(inv * inv)
    dv = inv / jnp.maximum(norm, 1e-12)

    e = jnp.where(iota == idx, dv, 0.0)
    enc_ref[...] = e

    q = jax.lax.dot_general(e, w, (((1,), (0,)), ((), ())),
                            preferred_element_type=jnp.float32)
    q_ref[...] = q

    diff = q - x
    ploss_ref[0, 0, 0] = jnp.sum(diff * diff)
    pprobs_ref[...] = jnp.sum(e, axis=0, keepdims=True)[None]


def _fin_body(pprobs_ref, ploss_ref, loss_ref, perp_ref):
    tot = jax.lax.fori_loop(
        0, _G, lambda j, acc: acc + ploss_ref[j, 0, 0], jnp.float32(0.0))
    m = tot / jnp.float32(_N * _D)
    loss_ref[0, 0] = 1.25 * m
    avg = jnp.sum(pprobs_ref[...], axis=0) / jnp.float32(_N)
    ent = jnp.sum(avg * jnp.log(avg + 1e-10))
    perp_ref[0, 0] = jnp.exp(-ent)


def kernel(inputs, weight, n=1):
    x = jnp.transpose(inputs, (0, 2, 3, 1))
    flat = x.reshape(_N, _D)
    shift = (jnp.asarray(n, jnp.float32) - 1.0).reshape(1, 1)

    wn = pl.pallas_call(
        _wn_body,
        out_shape=jax.ShapeDtypeStruct((1, _K), jnp.float32),
    )(weight)

    enc, qf, pprobs, ploss = pl.pallas_call(
        _vq_body,
        grid=(_G,),
        in_specs=[
            pl.BlockSpec((_R, _D), lambda i: (i, 0)),
            pl.BlockSpec((_K, _D), lambda i: (0, 0)),
            pl.BlockSpec((1, _K), lambda i: (0, 0)),
            pl.BlockSpec(memory_space=pltpu.SMEM),
        ],
        out_specs=[
            pl.BlockSpec((_R, _K), lambda i: (i, 0)),
            pl.BlockSpec((_R, _D), lambda i: (i, 0)),
            pl.BlockSpec((1, 1, _K), lambda i: (i, 0, 0)),
            pl.BlockSpec((1, 1, 1), lambda i: (i, 0, 0),
                         memory_space=pltpu.SMEM),
        ],
        out_shape=[
            jax.ShapeDtypeStruct((_N, _K), jnp.float32),
            jax.ShapeDtypeStruct((_N, _D), jnp.float32),
            jax.ShapeDtypeStruct((_G, 1, _K), jnp.float32),
            jax.ShapeDtypeStruct((_G, 1, 1), jnp.float32),
        ],
        compiler_params=pltpu.CompilerParams(
            dimension_semantics=("parallel",),
        ),
    )(flat, weight, wn, shift)

    loss, perp = pl.pallas_call(
        _fin_body,
        in_specs=[
            pl.BlockSpec((_G, 1, _K), lambda: (0, 0, 0)),
            pl.BlockSpec(memory_space=pltpu.SMEM),
        ],
        out_specs=[
            pl.BlockSpec(memory_space=pltpu.SMEM),
            pl.BlockSpec(memory_space=pltpu.SMEM),
        ],
        out_shape=[
            jax.ShapeDtypeStruct((1, 1), jnp.float32),
            jax.ShapeDtypeStruct((1, 1), jnp.float32),
        ],
    )(pprobs, ploss)

    quantized = jnp.transpose(qf.reshape(x.shape), (0, 3, 1, 2))
    return (loss[0, 0], quantized, perp[0, 0], enc)


# speedup vs baseline: 1.0372x; 1.0372x over previous
import jax
import jax.numpy as jnp
from jax.experimental import pallas as pl
from jax.experimental.pallas import tpu as pltpu

_K = 8192
_D = 32
_N = 4096
_R = 128
_G = _N // _R


def _wn_body(w_ref, wn_ref):
    w = w_ref[...]
    wn_ref[...] = jnp.sum(w * w, axis=1)[None, :]


def _vq_body(x_ref, w_ref, wn_ref, s_ref, enc_ref, q_ref, pprobs_ref,
             ploss_ref):
    x = x_ref[...]
    w = w_ref[...]
    rn = jnp.sum(x * x, axis=1, keepdims=True)
    wn = wn_ref[...]
    mm = jax.lax.dot_general(x, w, (((1,), (1,)), ((), ())),
                             preferred_element_type=jnp.float32)
    d = (rn + wn) - 2.0 * mm

    md = jnp.min(d, axis=1, keepdims=True)
    iota = jax.lax.broadcasted_iota(jnp.int32, d.shape, 1)
    idx = jnp.argmin(d, axis=1).astype(jnp.int32)[:, None]

    mds = md + s_ref[0, 0]
    inv = 1.0 / mds
    norm = jnp.sqrt(inv * inv)
    dv = inv / jnp.maximum(norm, 1e-12)

    e = jnp.where(iota == idx, dv, 0.0)
    enc_ref[...] = e

    q = jax.lax.dot_general(e, w, (((1,), (0,)), ((), ())),
                            preferred_element_type=jnp.float32)
    q_ref[...] = q

    diff = q - x
    ploss_ref[0, 0, 0] = jnp.sum(diff * diff)
    pprobs_ref[...] = jnp.sum(e, axis=0, keepdims=True)[None]


def _fin_body(pprobs_ref, ploss_ref, loss_ref, perp_ref):
    tot = jax.lax.fori_loop(
        0, _G, lambda j, acc: acc + ploss_ref[j, 0, 0], jnp.float32(0.0))
    m = tot / jnp.float32(_N * _D)
    loss_ref[0, 0] = 1.25 * m
    avg = jnp.sum(pprobs_ref[...], axis=0) / jnp.float32(_N)
    ent = jnp.sum(avg * jnp.log(avg + 1e-10))
    perp_ref[0, 0] = jnp.exp(-ent)


def kernel(inputs, weight, n=1):
    x = jnp.transpose(inputs, (0, 2, 3, 1))
    flat = x.reshape(_N, _D)
    shift = (jnp.asarray(n, jnp.float32) - 1.0).reshape(1, 1)

    wn = pl.pallas_call(
        _wn_body,
        out_shape=jax.ShapeDtypeStruct((1, _K), jnp.float32),
    )(weight)

    enc, qf, pprobs, ploss = pl.pallas_call(
        _vq_body,
        grid=(_G,),
        in_specs=[
            pl.BlockSpec((_R, _D), lambda i: (i, 0)),
            pl.BlockSpec((_K, _D), lambda i: (0, 0)),
            pl.BlockSpec((1, _K), lambda i: (0, 0)),
            pl.BlockSpec(memory_space=pltpu.SMEM),
        ],
        out_specs=[
            pl.BlockSpec((_R, _K), lambda i: (i, 0)),
            pl.BlockSpec((_R, _D), lambda i: (i, 0)),
            pl.BlockSpec((1, 1, _K), lambda i: (i, 0, 0)),
            pl.BlockSpec((1, 1, 1), lambda i: (i, 0, 0),
                         memory_space=pltpu.SMEM),
        ],
        out_shape=[
            jax.ShapeDtypeStruct((_N, _K), jnp.float32),
            jax.ShapeDtypeStruct((_N, _D), jnp.float32),
            jax.ShapeDtypeStruct((_G, 1, _K), jnp.float32),
            jax.ShapeDtypeStruct((_G, 1, 1), jnp.float32),
        ],
        compiler_params=pltpu.CompilerParams(
            dimension_semantics=("parallel",),
        ),
    )(flat, weight, wn, shift)

    loss, perp = pl.pallas_call(
        _fin_body,
        in_specs=[
            pl.BlockSpec((_G, 1, _K), lambda: (0, 0, 0)),
            pl.BlockSpec(memory_space=pltpu.SMEM),
        ],
        out_specs=[
            pl.BlockSpec(memory_space=pltpu.SMEM),
            pl.BlockSpec(memory_space=pltpu.SMEM),
        ],
        out_shape=[
            jax.ShapeDtypeStruct((1, 1), jnp.float32),
            jax.ShapeDtypeStruct((1, 1), jnp.float32),
        ],
    )(pprobs, ploss)

    quantized = jnp.transpose(qf.reshape(x.shape), (0, 3, 1, 2))
    return (loss[0, 0], quantized, perp[0, 0], enc)
